# Initial kernel scaffold; baseline (speedup 1.0000x reference)
import jax
import jax.numpy as jnp
from jax.experimental import pallas as pl


def kernel(x):
    raise NotImplementedError("write your pallas kernel here")


# speedup vs baseline: 1.9874x; 1.9874x over previous
import functools

import jax
import jax.numpy as jnp
from jax import lax
from jax.experimental import pallas as pl
from jax.experimental.pallas import tpu as pltpu
from jax.experimental.pallas import tpu_sc as plsc

_NC = 2
_NS = 16
_NW = _NC * _NS
_E = 32
_K = 4
_ROWS = 4 * 8192
_ROWS_PER_W = _ROWS // _NW
_LANES = 16


def _sc_partial_counts(xt):
    mesh = plsc.VectorSubcoreMesh(core_axis_name="c", subcore_axis_name="s")

    @functools.partial(
        pl.kernel,
        mesh=mesh,
        out_type=jax.ShapeDtypeStruct((_NW, _LANES), jnp.int32),
        scratch_types=[
            pltpu.VMEM((_E, _ROWS_PER_W), jnp.float32),
            pltpu.VMEM((_LANES,), jnp.int32),
        ],
    )
    def k(xt_hbm, out_hbm, xv, accv):
        wid = lax.axis_index("s") * _NC + lax.axis_index("c")
        base = wid * _ROWS_PER_W
        pltpu.sync_copy(xt_hbm.at[:, pl.ds(base, _ROWS_PER_W)], xv)

        neg = jnp.full((_LANES,), -jnp.inf, jnp.float32)
        zero = jnp.zeros((_LANES,), jnp.int32)
        one = jnp.ones((_LANES,), jnp.int32)

        def body(b, acc):
            b0 = b * _LANES
            t1 = neg
            t2 = neg
            t3 = neg
            t4 = neg
            for c in range(_E):
                v = xv[c, pl.ds(b0, _LANES)]
                m1 = jnp.minimum(t1, v)
                t1 = jnp.maximum(t1, v)
                m2 = jnp.minimum(t2, m1)
                t2 = jnp.maximum(t2, m1)
                m3 = jnp.minimum(t3, m2)
                t3 = jnp.maximum(t3, m2)
                t4 = jnp.maximum(t4, m3)
            for t in (t1, t2, t3, t4):
                acc = acc + jnp.where(t > neg, one, zero)
            return acc

        acc = lax.fori_loop(0, _ROWS_PER_W // _LANES, body, zero)
        accv[...] = acc
        pltpu.sync_copy(accv, out_hbm.at[wid])

    return k(xt)


def _tc_total(parts):
    def k(p_ref, o_ref):
        o_ref[0, 0] = jnp.sum(p_ref[...])

    return pl.pallas_call(
        k,
        out_shape=jax.ShapeDtypeStruct((1, 1), jnp.int32),
        out_specs=pl.BlockSpec(memory_space=pltpu.SMEM),
    )(parts)


def kernel(x):
    xt = x.reshape(_ROWS, _E).T
    parts = _sc_partial_counts(xt)
    return _tc_total(parts)[0, 0]
